# baseline (speedup 1.0000x reference)
import jax
import jax.numpy as jnp
from jax.experimental import pallas as pl
from jax.experimental.pallas import tpu as pltpu

DIM = 2048
KNOWLEDGE_DIM = 512
HALF = KNOWLEDGE_DIM // 2
NUM_KEYS = 1024
NUM_CANDIDATES = 32
NUM_EXTRACT = 36
IMIN = -2**31


def _branch_topk(s, inv_iota):
    r = s.shape[0]
    bits = jax.lax.bitcast_convert_type(s, jnp.int32)
    fkey = jax.lax.bitcast_convert_type(
        (bits & jnp.int32(-1024)) | inv_iota, jnp.float32)

    ch = [fkey[:, c * 128:(c + 1) * 128] for c in range(8)]
    for a, b in ((0, 1), (2, 3), (4, 5), (6, 7),
                 (0, 2), (1, 3), (4, 6), (5, 7),
                 (1, 2), (5, 6),
                 (0, 4), (1, 5), (2, 6), (3, 7),
                 (2, 4), (3, 5),
                 (1, 2), (3, 4), (5, 6)):
        hi_, lo_ = jnp.maximum(ch[a], ch[b]), jnp.minimum(ch[a], ch[b])
        ch[a], ch[b] = hi_, lo_

    iota128 = jax.lax.broadcasted_iota(jnp.int32, (r, 128), 1)
    ms = []
    for _ in range(NUM_EXTRACT):
        m = jnp.max(ch[0], axis=1, keepdims=True)
        ms.append(m)
        mb = jax.lax.bitcast_convert_type(m, jnp.int32)
        lam = (jnp.int32(1023) - (mb & jnp.int32(1023))) & jnp.int32(127)
        hit = iota128 == lam
        for j in range(7):
            ch[j] = jnp.where(hit, ch[j + 1], ch[j])
        ch[7] = jnp.where(hit, -jnp.inf, ch[7])
    mbits = jax.lax.bitcast_convert_type(
        jnp.concatenate(ms, axis=1), jnp.int32)
    pos = jnp.int32(1023) - (mbits & jnp.int32(1023))

    hi = jax.lax.shift_right_logical(pos, 7)
    lo = pos & jnp.int32(127)
    sv = jnp.zeros((r, NUM_EXTRACT), jnp.float32)
    for c in range(NUM_KEYS // 128):
        g = jnp.take_along_axis(s[:, c * 128:(c + 1) * 128], lo, axis=1)
        sv = jnp.where(hi == c, g, sv)

    iota_e = jax.lax.broadcasted_iota(jnp.int32, (r, NUM_EXTRACT), 1)
    fk2 = jax.lax.bitcast_convert_type(
        (jax.lax.bitcast_convert_type(sv, jnp.int32) & jnp.int32(-64))
        | (jnp.int32(63) - iota_e), jnp.float32)
    lps = []
    for _ in range(NUM_CANDIDATES):
        m = jnp.max(fk2, axis=1, keepdims=True)
        fk2 = jnp.where(fk2 == m, -jnp.inf, fk2)
        lps.append(m)
    lbits = jax.lax.bitcast_convert_type(
        jnp.concatenate(lps, axis=1), jnp.int32)
    lp = jnp.int32(63) - (lbits & jnp.int32(63))
    vals_c = jnp.take_along_axis(sv, lp, axis=1)
    idx = jnp.take_along_axis(pos, lp, axis=1)
    return [vals_c[:, t:t + 1] for t in range(NUM_CANDIDATES)], vals_c, idx


def _body(x_ref, w_ref, keys_ref, inv_iota_ref, idx_ref, scr_ref,
          s1_sc, s2_sc):
    r = x_ref.shape[0]
    s1 = s1_sc[...]
    s2 = s2_sc[...]

    inv_iota = inv_iota_ref[...]
    v1, _, i1 = _branch_topk(s1, inv_iota)
    _, v2c, i2c = _branch_topk(s2, inv_iota)

    comb_s, comb_i = [], []
    ncand = 0
    for t in range(NUM_CANDIDATES):
        c = NUM_CANDIDATES // (t + 1)
        comb_s.append(v1[t] + v2c[:, :c])
        comb_i.append(i1[:, t:t + 1] * NUM_KEYS + i2c[:, :c])
        ncand += c
    npad = 128 - ncand
    comb_s.append(jnp.full((r, npad), -jnp.inf, jnp.float32))
    comb_i.append(jnp.zeros((r, npad), jnp.int32))
    comb_s = jnp.concatenate(comb_s, axis=1)
    comb_i = jnp.concatenate(comb_i, axis=1)

    iota_c = jax.lax.broadcasted_iota(jnp.int32, (r, 128), 1)
    vals, poss = [], []
    s = comb_s
    for _ in range(NUM_CANDIDATES):
        m = jnp.max(s, axis=1, keepdims=True)
        pos = jnp.min(jnp.where(s == m, iota_c, 128), axis=1, keepdims=True)
        vals.append(m)
        poss.append(pos)
        s = jnp.where(iota_c == pos, -jnp.inf, s)

    top_s = jnp.concatenate(vals, axis=1)
    top_i = jnp.take_along_axis(comb_i, jnp.concatenate(poss, axis=1), axis=1)

    mx = jnp.max(top_s, axis=1, keepdims=True)
    e = jnp.exp(top_s - mx)
    p = e / jnp.sum(e, axis=1, keepdims=True)

    idx_ref[...] = top_i
    scr_ref[...] = p

    q = jnp.dot(x_ref[...], w_ref[...], preferred_element_type=jnp.float32)
    keys = keys_ref[...]
    s1_sc[...] = jnp.dot(q[:, :HALF], keys[0],
                         preferred_element_type=jnp.float32)
    s2_sc[...] = jnp.dot(q[:, HALF:], keys[1],
                         preferred_element_type=jnp.float32)


@jax.jit
def kernel(x, W, keys):
    bsz, seq_len, d = x.shape
    n = bsz * seq_len
    xf = x.reshape(n, d)
    keys_t = jnp.transpose(keys, (0, 2, 1))
    inv_iota = (jnp.int32(NUM_KEYS - 1)
                - jax.lax.broadcasted_iota(jnp.int32, (1, NUM_KEYS), 1))

    r = 512 if n % 512 == 0 else n
    nblk = n // r

    idx, scr = pl.pallas_call(
        _body,
        grid=(nblk + 1,),
        in_specs=[
            pl.BlockSpec((r, d), lambda i: (jnp.minimum(i, nblk - 1), 0)),
            pl.BlockSpec((d, KNOWLEDGE_DIM), lambda i: (0, 0)),
            pl.BlockSpec((2, HALF, NUM_KEYS), lambda i: (0, 0, 0)),
            pl.BlockSpec((1, NUM_KEYS), lambda i: (0, 0)),
        ],
        out_specs=[
            pl.BlockSpec((r, NUM_CANDIDATES), lambda i: (jnp.maximum(i - 1, 0), 0)),
            pl.BlockSpec((r, NUM_CANDIDATES), lambda i: (jnp.maximum(i - 1, 0), 0)),
        ],
        scratch_shapes=[
            pltpu.VMEM((r, NUM_KEYS), jnp.float32),
            pltpu.VMEM((r, NUM_KEYS), jnp.float32),
        ],
        out_shape=[
            jax.ShapeDtypeStruct((n, NUM_CANDIDATES), jnp.int32),
            jax.ShapeDtypeStruct((n, NUM_CANDIDATES), jnp.float32),
        ],
    )(xf, W, keys_t, inv_iota)

    return (idx.reshape(bsz, seq_len, NUM_CANDIDATES),
            scr.reshape(bsz, seq_len, NUM_CANDIDATES))


# speedup vs baseline: 1.0543x; 1.0543x over previous
import jax
import jax.numpy as jnp
from jax.experimental import pallas as pl

DIM = 2048
KNOWLEDGE_DIM = 512
HALF = KNOWLEDGE_DIM // 2
NUM_KEYS = 1024
NUM_CANDIDATES = 32
NUM_EXTRACT = 36
IMIN = -2**31


def _branch_topk(s, inv_iota):
    r = s.shape[0]
    bits = jax.lax.bitcast_convert_type(s, jnp.int32)
    fkey = jax.lax.bitcast_convert_type(
        (bits & jnp.int32(-1024)) | inv_iota, jnp.float32)

    ch = [fkey[:, c * 128:(c + 1) * 128] for c in range(8)]
    for a, b in ((0, 1), (2, 3), (4, 5), (6, 7),
                 (0, 2), (1, 3), (4, 6), (5, 7),
                 (1, 2), (5, 6),
                 (0, 4), (1, 5), (2, 6), (3, 7),
                 (2, 4), (3, 5),
                 (1, 2), (3, 4), (5, 6)):
        hi_, lo_ = jnp.maximum(ch[a], ch[b]), jnp.minimum(ch[a], ch[b])
        ch[a], ch[b] = hi_, lo_

    iota128 = jax.lax.broadcasted_iota(jnp.int32, (r, 128), 1)
    ms = []
    for _ in range(NUM_EXTRACT):
        m = jnp.max(ch[0], axis=1, keepdims=True)
        ms.append(m)
        mb = jax.lax.bitcast_convert_type(m, jnp.int32)
        lam = (jnp.int32(1023) - (mb & jnp.int32(1023))) & jnp.int32(127)
        hit = iota128 == lam
        for j in range(7):
            ch[j] = jnp.where(hit, ch[j + 1], ch[j])
        ch[7] = jnp.where(hit, -jnp.inf, ch[7])
    mbits = jax.lax.bitcast_convert_type(
        jnp.concatenate(ms, axis=1), jnp.int32)
    pos = jnp.int32(1023) - (mbits & jnp.int32(1023))

    hi = jax.lax.shift_right_logical(pos, 7)
    lo = pos & jnp.int32(127)
    sv = jnp.zeros((r, NUM_EXTRACT), jnp.float32)
    for c in range(NUM_KEYS // 128):
        g = jnp.take_along_axis(s[:, c * 128:(c + 1) * 128], lo, axis=1)
        sv = jnp.where(hi == c, g, sv)

    iota_e = jax.lax.broadcasted_iota(jnp.int32, (r, NUM_EXTRACT), 1)
    fk2 = jax.lax.bitcast_convert_type(
        (jax.lax.bitcast_convert_type(sv, jnp.int32) & jnp.int32(-64))
        | (jnp.int32(63) - iota_e), jnp.float32)
    lps = []
    for _ in range(NUM_CANDIDATES):
        m = jnp.max(fk2, axis=1, keepdims=True)
        fk2 = jnp.where(fk2 == m, -jnp.inf, fk2)
        lps.append(m)
    lbits = jax.lax.bitcast_convert_type(
        jnp.concatenate(lps, axis=1), jnp.int32)
    lp = jnp.int32(63) - (lbits & jnp.int32(63))
    vals_c = jnp.take_along_axis(sv, lp, axis=1)
    idx = jnp.take_along_axis(pos, lp, axis=1)
    return [vals_c[:, t:t + 1] for t in range(NUM_CANDIDATES)], vals_c, idx


def _body(x_ref, w_ref, keys_ref, inv_iota_ref, idx_ref, scr_ref):
    r = x_ref.shape[0]
    q = jnp.dot(x_ref[...], w_ref[...], preferred_element_type=jnp.float32)
    keys = keys_ref[...]
    s1 = jnp.dot(q[:, :HALF], keys[0], preferred_element_type=jnp.float32)
    s2 = jnp.dot(q[:, HALF:], keys[1], preferred_element_type=jnp.float32)

    inv_iota = inv_iota_ref[...]
    _, vc, ic = _branch_topk(jnp.concatenate([s1, s2], axis=0), inv_iota)
    v1c, v2c = vc[:r], vc[r:]
    i1, i2c = ic[:r], ic[r:]
    v1 = [v1c[:, t:t + 1] for t in range(NUM_CANDIDATES)]

    comb_s, comb_i = [], []
    ncand = 0
    for t in range(NUM_CANDIDATES):
        c = NUM_CANDIDATES // (t + 1)
        comb_s.append(v1[t] + v2c[:, :c])
        comb_i.append(i1[:, t:t + 1] * NUM_KEYS + i2c[:, :c])
        ncand += c
    npad = 128 - ncand
    comb_s.append(jnp.full((r, npad), -jnp.inf, jnp.float32))
    comb_i.append(jnp.zeros((r, npad), jnp.int32))
    comb_s = jnp.concatenate(comb_s, axis=1)
    comb_i = jnp.concatenate(comb_i, axis=1)

    iota_c = jax.lax.broadcasted_iota(jnp.int32, (r, 128), 1)
    vals, poss = [], []
    s = comb_s
    for _ in range(NUM_CANDIDATES):
        m = jnp.max(s, axis=1, keepdims=True)
        pos = jnp.min(jnp.where(s == m, iota_c, 128), axis=1, keepdims=True)
        vals.append(m)
        poss.append(pos)
        s = jnp.where(iota_c == pos, -jnp.inf, s)

    top_s = jnp.concatenate(vals, axis=1)
    top_i = jnp.take_along_axis(comb_i, jnp.concatenate(poss, axis=1), axis=1)

    mx = jnp.max(top_s, axis=1, keepdims=True)
    e = jnp.exp(top_s - mx)
    p = e / jnp.sum(e, axis=1, keepdims=True)

    idx_ref[...] = top_i
    scr_ref[...] = p


@jax.jit
def kernel(x, W, keys):
    bsz, seq_len, d = x.shape
    n = bsz * seq_len
    xf = x.reshape(n, d)
    keys_t = jnp.transpose(keys, (0, 2, 1))
    inv_iota = (jnp.int32(NUM_KEYS - 1)
                - jax.lax.broadcasted_iota(jnp.int32, (1, NUM_KEYS), 1))

    r = 512 if n % 512 == 0 else n
    grid = n // r

    idx, scr = pl.pallas_call(
        _body,
        grid=(grid,),
        in_specs=[
            pl.BlockSpec((r, d), lambda i: (i, 0)),
            pl.BlockSpec((d, KNOWLEDGE_DIM), lambda i: (0, 0)),
            pl.BlockSpec((2, HALF, NUM_KEYS), lambda i: (0, 0, 0)),
            pl.BlockSpec((1, NUM_KEYS), lambda i: (0, 0)),
        ],
        out_specs=[
            pl.BlockSpec((r, NUM_CANDIDATES), lambda i: (i, 0)),
            pl.BlockSpec((r, NUM_CANDIDATES), lambda i: (i, 0)),
        ],
        out_shape=[
            jax.ShapeDtypeStruct((n, NUM_CANDIDATES), jnp.int32),
            jax.ShapeDtypeStruct((n, NUM_CANDIDATES), jnp.float32),
        ],
    )(xf, W, keys_t, inv_iota)

    return (idx.reshape(bsz, seq_len, NUM_CANDIDATES),
            scr.reshape(bsz, seq_len, NUM_CANDIDATES))


# speedup vs baseline: 1.0595x; 1.0050x over previous
import jax
import jax.numpy as jnp
from jax.experimental import pallas as pl

DIM = 2048
KNOWLEDGE_DIM = 512
HALF = KNOWLEDGE_DIM // 2
NUM_KEYS = 1024
NUM_CANDIDATES = 32
NUM_EXTRACT = 36


def _branch_topk(s, inv_iota):
    r = s.shape[0]
    bits = jax.lax.bitcast_convert_type(s, jnp.int32)
    fkey = jax.lax.bitcast_convert_type(
        (bits & jnp.int32(-1024)) | inv_iota, jnp.float32)

    ch = [fkey[:, c * 128:(c + 1) * 128] for c in range(8)]
    for a, b in ((0, 1), (2, 3), (4, 5), (6, 7),
                 (0, 2), (1, 3), (4, 6), (5, 7),
                 (1, 2), (5, 6),
                 (0, 4), (1, 5), (2, 6), (3, 7),
                 (2, 4), (3, 5),
                 (1, 2), (3, 4), (5, 6)):
        hi_, lo_ = jnp.maximum(ch[a], ch[b]), jnp.minimum(ch[a], ch[b])
        ch[a], ch[b] = hi_, lo_

    iota128 = jax.lax.broadcasted_iota(jnp.int32, (r, 128), 1)
    ms = []
    for _ in range(NUM_EXTRACT):
        m = jnp.max(ch[0], axis=1, keepdims=True)
        ms.append(m)
        mb = jax.lax.bitcast_convert_type(m, jnp.int32)
        lam = (jnp.int32(1023) - (mb & jnp.int32(1023))) & jnp.int32(127)
        hit = iota128 == lam
        for j in range(7):
            ch[j] = jnp.where(hit, ch[j + 1], ch[j])
        ch[7] = jnp.where(hit, -jnp.inf, ch[7])
    mbits = jax.lax.bitcast_convert_type(
        jnp.concatenate(ms, axis=1), jnp.int32)
    pos = jnp.int32(1023) - (mbits & jnp.int32(1023))

    hi = jax.lax.shift_right_logical(pos, 7)
    lo = pos & jnp.int32(127)
    sv = jnp.zeros((r, NUM_EXTRACT), jnp.float32)
    for c in range(NUM_KEYS // 128):
        g = jnp.take_along_axis(s[:, c * 128:(c + 1) * 128], lo, axis=1)
        sv = jnp.where(hi == c, g, sv)

    iota_e = jax.lax.broadcasted_iota(jnp.int32, (r, NUM_EXTRACT), 1)
    fk2 = jax.lax.bitcast_convert_type(
        (jax.lax.bitcast_convert_type(sv, jnp.int32) & jnp.int32(-64))
        | (jnp.int32(63) - iota_e), jnp.float32)
    lps = []
    for _ in range(NUM_CANDIDATES):
        m = jnp.max(fk2, axis=1, keepdims=True)
        fk2 = jnp.where(fk2 == m, -jnp.inf, fk2)
        lps.append(m)
    lbits = jax.lax.bitcast_convert_type(
        jnp.concatenate(lps, axis=1), jnp.int32)
    lp = jnp.int32(63) - (lbits & jnp.int32(63))
    vals_c = jnp.take_along_axis(sv, lp, axis=1)
    idx = jnp.take_along_axis(pos, lp, axis=1)
    return [vals_c[:, t:t + 1] for t in range(NUM_CANDIDATES)], vals_c, idx


def _body(x_ref, w_ref, keys_ref, inv_iota_ref, idx_ref, scr_ref):
    r = x_ref.shape[0]
    q = jnp.dot(x_ref[...], w_ref[...], preferred_element_type=jnp.float32)
    keys = keys_ref[...]
    s1 = jnp.dot(q[:, :HALF], keys[0], preferred_element_type=jnp.float32)
    s2 = jnp.dot(q[:, HALF:], keys[1], preferred_element_type=jnp.float32)

    inv_iota = inv_iota_ref[...]
    v1, _, i1 = _branch_topk(s1, inv_iota)
    _, v2c, i2c = _branch_topk(s2, inv_iota)

    comb_s, comb_i = [], []
    ncand = 0
    for t in range(NUM_CANDIDATES):
        c = NUM_CANDIDATES // (t + 1)
        comb_s.append(v1[t] + v2c[:, :c])
        comb_i.append(i1[:, t:t + 1] * NUM_KEYS + i2c[:, :c])
        ncand += c
    npad = 128 - ncand
    comb_s.append(jnp.full((r, npad), -jnp.inf, jnp.float32))
    comb_i.append(jnp.zeros((r, npad), jnp.int32))
    comb_s = jnp.concatenate(comb_s, axis=1)
    comb_i = jnp.concatenate(comb_i, axis=1)

    iota_c = jax.lax.broadcasted_iota(jnp.int32, (r, 128), 1)
    vals, poss = [], []
    s = comb_s
    for _ in range(NUM_CANDIDATES):
        m = jnp.max(s, axis=1, keepdims=True)
        pos = jnp.min(jnp.where(s == m, iota_c, 128), axis=1, keepdims=True)
        vals.append(m)
        poss.append(pos)
        s = jnp.where(iota_c == pos, -jnp.inf, s)

    top_s = jnp.concatenate(vals, axis=1)
    top_i = jnp.take_along_axis(comb_i, jnp.concatenate(poss, axis=1), axis=1)

    mx = jnp.max(top_s, axis=1, keepdims=True)
    e = jnp.exp(top_s - mx)
    p = e / jnp.sum(e, axis=1, keepdims=True)

    idx_ref[...] = top_i
    scr_ref[...] = p


@jax.jit
def kernel(x, W, keys):
    bsz, seq_len, d = x.shape
    n = bsz * seq_len
    xf = x.reshape(n, d)
    keys_t = jnp.transpose(keys, (0, 2, 1))
    inv_iota = (jnp.int32(NUM_KEYS - 1)
                - jax.lax.broadcasted_iota(jnp.int32, (1, NUM_KEYS), 1))

    r = 512 if n % 512 == 0 else n
    grid = n // r

    idx, scr = pl.pallas_call(
        _body,
        grid=(grid,),
        in_specs=[
            pl.BlockSpec((r, d), lambda i: (i, 0)),
            pl.BlockSpec((d, KNOWLEDGE_DIM), lambda i: (0, 0)),
            pl.BlockSpec((2, HALF, NUM_KEYS), lambda i: (0, 0, 0)),
            pl.BlockSpec((1, NUM_KEYS), lambda i: (0, 0)),
        ],
        out_specs=[
            pl.BlockSpec((r, NUM_CANDIDATES), lambda i: (i, 0)),
            pl.BlockSpec((r, NUM_CANDIDATES), lambda i: (i, 0)),
        ],
        out_shape=[
            jax.ShapeDtypeStruct((n, NUM_CANDIDATES), jnp.int32),
            jax.ShapeDtypeStruct((n, NUM_CANDIDATES), jnp.float32),
        ],
    )(xf, W, keys_t, inv_iota)

    return (idx.reshape(bsz, seq_len, NUM_CANDIDATES),
            scr.reshape(bsz, seq_len, NUM_CANDIDATES))
